# baseline (speedup 1.0000x reference)
import jax
import jax.numpy as jnp
from jax.experimental import pallas as pl
from jax.experimental.pallas import tpu as pltpu
from jax.experimental.pallas import tpu_sc as plsc

_K = 8192
_H = 256
_TM = 512
_TN = 1024
_BETA = 0.25
_GW = 128
_GROUPS = 2


def _argmin_body(z_ref, e_ref, idx_ref, bd_ref, e2_ref):
    n_chunks = _K // _TN

    @pl.when(pl.program_id(0) == 0)
    def _():
        e = e_ref[...]
        e2_ref[...] = jnp.sum(e * e, axis=1, keepdims=True).reshape(1, _K)

    z = z_ref[...]
    z2 = jnp.sum(z * z, axis=1, keepdims=True)
    zm2 = z * (-2.0)
    colf = jax.lax.broadcasted_iota(jnp.int32, (1, _TN), 1).astype(jnp.float32)
    bigf = jnp.float32(1e9)

    def mm(j):
        ec = e_ref[j * _TN:(j + 1) * _TN, :]
        return jax.lax.dot_general(
            zm2, ec, (((1,), (1,)), ((), ())),
            preferred_element_type=jnp.float32)

    pv = None
    pc = None
    pm2 = mm(0)
    for j in range(n_chunks):
        pm2_cur = pm2
        if j + 1 < n_chunks:
            pm2 = mm(j + 1)
        e2c = e2_ref[:, j * _TN:(j + 1) * _TN]
        dist = (z2 + e2c) + pm2_cur
        if j == 0:
            pv = dist
            pc = jnp.zeros((_TM, _TN), jnp.float32)
        else:
            lt = dist < pv
            pv = jnp.where(lt, dist, pv)
            pc = jnp.where(lt, jnp.float32(j), pc)

    cmin = jnp.min(pv, axis=1, keepdims=True)
    sel = jnp.where(pv == cmin, pc * _TN + colf, bigf)
    gidx = jnp.min(sel, axis=1, keepdims=True)
    idx_ref[...] = gidx.astype(jnp.int32).reshape(1, 1, _TM)
    bd_ref[...] = cmin.reshape(1, 1, _TM)


def _argmin_call(zf, emb):
    n_tiles = zf.shape[0] // _TM
    return pl.pallas_call(
        _argmin_body,
        grid=(n_tiles,),
        in_specs=[
            pl.BlockSpec((_TM, _H), lambda i: (i, 0)),
            pl.BlockSpec((_K, _H), lambda i: (0, 0)),
        ],
        out_specs=[
            pl.BlockSpec((1, 1, _TM), lambda i: (i, 0, 0)),
            pl.BlockSpec((1, 1, _TM), lambda i: (i, 0, 0)),
        ],
        out_shape=[
            jax.ShapeDtypeStruct((n_tiles, 1, _TM), jnp.int32),
            jax.ShapeDtypeStruct((n_tiles, 1, _TM), jnp.float32),
        ],
        scratch_shapes=[pltpu.VMEM((1, _K), jnp.float32)],
    )(zf, emb)


def _sc_gather(emb, idx3):
    n_tiles = idx3.shape[0]
    n = n_tiles * _TM
    wins_per_tile = _TM // _GW
    mesh = plsc.VectorSubcoreMesh(core_axis_name="core",
                                  subcore_axis_name="subcore")

    @pl.kernel(out_type=jax.ShapeDtypeStruct((n, _H), emb.dtype), mesh=mesh)
    def k(emb_hbm, i_hbm, o_hbm):
        def body(i_vmem, o_vmem):
            pltpu.sync_copy(emb_hbm.at[i_vmem.at[0, 0]], o_vmem)

        pltpu.emit_pipeline(
            body,
            grid=(n // _GW,),
            in_specs=[pl.BlockSpec(
                (1, 1, _GW),
                index_map=lambda i: (i // wins_per_tile, 0, i % wins_per_tile))],
            out_specs=[pl.BlockSpec((_GW, _H), index_map=lambda i: (i, 0))],
            core_axis_name=("core", "subcore"),
            dimension_semantics=(pltpu.PARALLEL,),
        )(i_hbm, o_hbm)

    return k(emb, idx3)


def kernel(z_e, embedding):
    zf = z_e.reshape(-1, _H)
    n = zf.shape[0]
    ng = n // _GROUPS
    parts = []
    bds = []
    for g in range(_GROUPS):
        idx, bd = _argmin_call(zf[g * ng:(g + 1) * ng], embedding)
        parts.append(_sc_gather(embedding, idx))
        bds.append(jnp.sum(bd))
    z_q = jnp.concatenate(parts, axis=0).reshape(z_e.shape)
    loss = (1.0 + _BETA) * (sum(bds) / zf.size)
    return z_q, loss


# speedup vs baseline: 1.2455x; 1.2455x over previous
import jax
import jax.numpy as jnp
from jax.experimental import pallas as pl
from jax.experimental.pallas import tpu as pltpu
from jax.experimental.pallas import tpu_sc as plsc

_K = 8192
_H = 256
_TM = 512
_TN = 1024
_BETA = 0.25
_GW = 128
_GROUPS = 2


def _argmin_body(z_ref, e_ref, idx_ref, bd_ref, e2_ref):
    n_chunks = _K // _TN

    @pl.when(pl.program_id(0) == 0)
    def _():
        e = e_ref[...]
        e2_ref[...] = jnp.sum(e * e, axis=1, keepdims=True).reshape(1, _K)

    z = z_ref[...]
    z2 = jnp.sum(z * z, axis=1, keepdims=True)
    zm2 = z * (-2.0)
    colf = jax.lax.broadcasted_iota(jnp.int32, (1, _TN), 1).astype(jnp.float32)
    bigf = jnp.float32(1e9)

    def mm(j):
        ec = e_ref[j * _TN:(j + 1) * _TN, :]
        return jax.lax.dot_general(
            zm2, ec, (((1,), (1,)), ((), ())),
            preferred_element_type=jnp.float32)

    pv = None
    pc = None
    pm2 = mm(0)
    for j in range(n_chunks):
        pm2_cur = pm2
        if j + 1 < n_chunks:
            pm2 = mm(j + 1)
        e2c = e2_ref[:, j * _TN:(j + 1) * _TN]
        dist = (z2 + e2c) + pm2_cur
        if j == 0:
            pv = dist
            pc = jnp.zeros((_TM, _TN), jnp.float32)
        else:
            pc = jnp.where(dist < pv, jnp.float32(j), pc)
            pv = jnp.minimum(pv, dist)

    cmin = jnp.min(pv, axis=1, keepdims=True)
    sel = jnp.where(pv == cmin, pc * _TN + colf, bigf)
    gidx = jnp.min(sel, axis=1, keepdims=True)
    idx_ref[...] = gidx.astype(jnp.int32).reshape(1, 1, _TM)
    bd_ref[...] = cmin.reshape(1, 1, _TM)


def _argmin_call(zf, emb):
    n_tiles = zf.shape[0] // _TM
    return pl.pallas_call(
        _argmin_body,
        grid=(n_tiles,),
        in_specs=[
            pl.BlockSpec((_TM, _H), lambda i: (i, 0)),
            pl.BlockSpec((_K, _H), lambda i: (0, 0)),
        ],
        out_specs=[
            pl.BlockSpec((1, 1, _TM), lambda i: (i, 0, 0)),
            pl.BlockSpec((1, 1, _TM), lambda i: (i, 0, 0)),
        ],
        out_shape=[
            jax.ShapeDtypeStruct((n_tiles, 1, _TM), jnp.int32),
            jax.ShapeDtypeStruct((n_tiles, 1, _TM), jnp.float32),
        ],
        scratch_shapes=[pltpu.VMEM((1, _K), jnp.float32)],
    )(zf, emb)


def _sc_gather(emb, idx3):
    n_tiles = idx3.shape[0]
    n = n_tiles * _TM
    wins_per_tile = _TM // _GW
    mesh = plsc.VectorSubcoreMesh(core_axis_name="core",
                                  subcore_axis_name="subcore")

    @pl.kernel(out_type=jax.ShapeDtypeStruct((n, _H), emb.dtype), mesh=mesh)
    def k(emb_hbm, i_hbm, o_hbm):
        def body(i_vmem, o_vmem):
            pltpu.sync_copy(emb_hbm.at[i_vmem.at[0, 0]], o_vmem)

        pltpu.emit_pipeline(
            body,
            grid=(n // _GW,),
            in_specs=[pl.BlockSpec(
                (1, 1, _GW),
                index_map=lambda i: (i // wins_per_tile, 0, i % wins_per_tile))],
            out_specs=[pl.BlockSpec((_GW, _H), index_map=lambda i: (i, 0))],
            core_axis_name=("core", "subcore"),
            dimension_semantics=(pltpu.PARALLEL,),
        )(i_hbm, o_hbm)

    return k(emb, idx3)


def kernel(z_e, embedding):
    zf = z_e.reshape(-1, _H)
    idx, bd = _argmin_call(zf, embedding)
    z_q = _sc_gather(embedding, idx).reshape(z_e.shape)
    loss = (1.0 + _BETA) * (jnp.sum(bd) / zf.size)
    return z_q, loss


# speedup vs baseline: 1.3723x; 1.1018x over previous
import jax
import jax.numpy as jnp
from jax.experimental import pallas as pl
from jax.experimental.pallas import tpu as pltpu
from jax.experimental.pallas import tpu_sc as plsc

_K = 8192
_H = 256
_TM = 512
_TN = 1024
_BETA = 0.25
_GW = 128
_GROUPS = 2


def _argmin_body(z_ref, e_ref, idx_ref, bd_ref, e2_ref):
    n_chunks = _K // _TN

    @pl.when(pl.program_id(0) == 0)
    def _():
        e = e_ref[...]
        e2_ref[...] = jnp.sum(e * e, axis=1, keepdims=True).reshape(1, _K)

    z = z_ref[...]
    z2 = jnp.sum(z * z, axis=1, keepdims=True)
    zm2 = z * (-2.0)
    colf = jax.lax.broadcasted_iota(jnp.int32, (1, _TN), 1).astype(jnp.float32)
    bigf = jnp.float32(1e9)

    def mm(j):
        ec = e_ref[j * _TN:(j + 1) * _TN, :]
        return jax.lax.dot_general(
            zm2, ec, (((1,), (1,)), ((), ())),
            preferred_element_type=jnp.float32)

    pv = None
    pc = None
    pm2 = mm(0)
    for j in range(n_chunks):
        pm2_cur = pm2
        if j + 1 < n_chunks:
            pm2 = mm(j + 1)
        e2c = e2_ref[:, j * _TN:(j + 1) * _TN]
        dist = e2c + pm2_cur
        if j == 0:
            pv = dist
            pc = jnp.zeros((_TM, _TN), jnp.float32)
        else:
            pc = jnp.where(dist < pv, jnp.float32(j), pc)
            pv = jnp.minimum(pv, dist)

    cmin = jnp.min(pv, axis=1, keepdims=True)
    sel = jnp.where(pv == cmin, pc * _TN + colf, bigf)
    gidx = jnp.min(sel, axis=1, keepdims=True)
    idx_ref[...] = gidx.astype(jnp.int32).reshape(1, 1, _TM)
    bd_ref[...] = (z2 + cmin).reshape(1, 1, _TM)


def _argmin_call(zf, emb):
    n_tiles = zf.shape[0] // _TM
    return pl.pallas_call(
        _argmin_body,
        grid=(n_tiles,),
        in_specs=[
            pl.BlockSpec((_TM, _H), lambda i: (i, 0)),
            pl.BlockSpec((_K, _H), lambda i: (0, 0)),
        ],
        out_specs=[
            pl.BlockSpec((1, 1, _TM), lambda i: (i, 0, 0)),
            pl.BlockSpec((1, 1, _TM), lambda i: (i, 0, 0)),
        ],
        out_shape=[
            jax.ShapeDtypeStruct((n_tiles, 1, _TM), jnp.int32),
            jax.ShapeDtypeStruct((n_tiles, 1, _TM), jnp.float32),
        ],
        scratch_shapes=[pltpu.VMEM((1, _K), jnp.float32)],
    )(zf, emb)


def _sc_gather(emb, idx3):
    n_tiles = idx3.shape[0]
    n = n_tiles * _TM
    wins_per_tile = _TM // _GW
    mesh = plsc.VectorSubcoreMesh(core_axis_name="core",
                                  subcore_axis_name="subcore")

    @pl.kernel(out_type=jax.ShapeDtypeStruct((n, _H), emb.dtype), mesh=mesh)
    def k(emb_hbm, i_hbm, o_hbm):
        def body(i_vmem, o_vmem):
            pltpu.sync_copy(emb_hbm.at[i_vmem.at[0, 0]], o_vmem)

        pltpu.emit_pipeline(
            body,
            grid=(n // _GW,),
            in_specs=[pl.BlockSpec(
                (1, 1, _GW),
                index_map=lambda i: (i // wins_per_tile, 0, i % wins_per_tile))],
            out_specs=[pl.BlockSpec((_GW, _H), index_map=lambda i: (i, 0))],
            core_axis_name=("core", "subcore"),
            dimension_semantics=(pltpu.PARALLEL,),
        )(i_hbm, o_hbm)

    return k(emb, idx3)


def kernel(z_e, embedding):
    zf = z_e.reshape(-1, _H)
    idx, bd = _argmin_call(zf, embedding)
    z_q = _sc_gather(embedding, idx).reshape(z_e.shape)
    loss = (1.0 + _BETA) * (jnp.sum(bd) / zf.size)
    return z_q, loss
